# Initial kernel scaffold; baseline (speedup 1.0000x reference)
import functools

import jax
import jax.numpy as jnp
from jax import lax
from jax.experimental import pallas as pl
from jax.experimental.pallas import tpu as pltpu
from jax.experimental.pallas import tpu_sc as plsc

N = 10000
D = 128
DH = D // 2
E = 320000
K = 10
ALPHA = 0.2
NT = 16
CHUNK = 128
EPT = 20480
NCH = EPT // CHUNK
EPAD = EPT * NT
RPT = N // NT
RCH = 125
NRC = RPT // RCH


def _body(e_hbm, src_hbm, dst_hbm, w_hbm, out_hbm,
          xa, xb, ha, src_v, dst_v, w_v, gbuf, bounce):
  c = lax.axis_index("c")
  s = lax.axis_index("s")
  base = s * RPT

  pltpu.sync_copy(src_hbm.at[s], src_v)
  pltpu.sync_copy(dst_hbm.at[s], dst_v)
  pltpu.sync_copy(w_hbm.at[s], w_v)

  def _scale_w(j):
    for q in range(8):
      w_v[j, pl.ds(q * 16, 16)] = w_v[j, pl.ds(q * 16, 16)] * (1.0 - ALPHA)
  lax.fori_loop(0, NCH, lambda j, _: (_scale_w(j), None)[1], None)

  for i in range(NRC):
    r0 = base + i * RCH
    pltpu.sync_copy(e_hbm.at[c, pl.ds(r0, RCH)], bounce)
    pltpu.sync_copy(bounce, xa.at[pl.ds(r0, RCH)])

    def _alpha_row(r):
      for q in range(4):
        bounce[r, pl.ds(q * 16, 16)] = bounce[r, pl.ds(q * 16, 16)] * ALPHA
    lax.fori_loop(0, RCH, lambda r, _: (_alpha_row(r), None)[1], None)
    pltpu.sync_copy(bounce, ha.at[pl.ds(r0, RCH)])

  plsc.subcore_barrier()

  for it in range(K):
    in_buf, out_buf = (xa, xb) if it % 2 == 0 else (xb, xa)

    for i in range(NRC):
      r0 = base + i * RCH
      pltpu.sync_copy(ha.at[pl.ds(r0, RCH)], bounce)
      pltpu.sync_copy(bounce, out_buf.at[pl.ds(r0, RCH)])
    plsc.subcore_barrier()

    def _chunk(j):
      pltpu.sync_copy(in_buf.at[src_v.at[j]], gbuf)

      def _edge(e):
        w = w_v[j, e]
        for q in range(4):
          gbuf[e, pl.ds(q * 16, 16)] = gbuf[e, pl.ds(q * 16, 16)] * w
      lax.fori_loop(0, CHUNK, lambda e, _: (_edge(e), None)[1], None)

      pltpu.sync_copy(gbuf, out_buf.at[dst_v.at[j]], add=True)
    lax.fori_loop(0, NCH, lambda j, _: (_chunk(j), None)[1], None)
    plsc.subcore_barrier()

  for i in range(NRC):
    r0 = base + i * RCH
    pltpu.sync_copy(xa.at[pl.ds(r0, RCH)], bounce)
    pltpu.sync_copy(bounce, out_hbm.at[c, pl.ds(r0, RCH)])


_sc_call = pl.kernel(
    _body,
    out_type=jax.ShapeDtypeStruct((2, N, DH), jnp.float32),
    mesh=plsc.VectorSubcoreMesh(core_axis_name="c", subcore_axis_name="s"),
    scratch_types=[
        pltpu.VMEM_SHARED((N, DH), jnp.float32),
        pltpu.VMEM_SHARED((N, DH), jnp.float32),
        pltpu.VMEM_SHARED((N, DH), jnp.float32),
        pltpu.VMEM((NCH, CHUNK), jnp.int32),
        pltpu.VMEM((NCH, CHUNK), jnp.int32),
        pltpu.VMEM((NCH, CHUNK), jnp.float32),
        pltpu.VMEM((CHUNK, DH), jnp.float32),
        pltpu.VMEM((RCH, DH), jnp.float32),
    ],
)


@jax.jit
def kernel(ent_embed, edge_index, edge_weight):
  dst = edge_index[0].astype(jnp.int32)
  src = edge_index[1].astype(jnp.int32)
  w = edge_weight.astype(jnp.float32)

  pad = EPAD - E
  src_p = jnp.pad(src, (0, pad)).reshape(NT, NCH, CHUNK)
  dst_p = jnp.pad(dst, (0, pad)).reshape(NT, NCH, CHUNK)
  w_p = jnp.pad(w, (0, pad)).reshape(NT, NCH, CHUNK)

  e_halves = ent_embed.reshape(N, 2, DH).transpose(1, 0, 2)

  out = _sc_call(e_halves, src_p, dst_p, w_p)
  return out.transpose(1, 0, 2).reshape(N, D)


# speedup vs baseline: 3.0062x; 3.0062x over previous
import jax
import jax.numpy as jnp
from jax import lax
from jax.experimental import pallas as pl
from jax.experimental.pallas import tpu as pltpu
from jax.experimental.pallas import tpu_sc as plsc

N = 10000
NPAD = 10240
D = 128
DH = D // 2
E = 320000
K = 10
ALPHA = 0.2
NT = 16
CHUNK = 128
EPT = 20480
NCH = EPT // CHUNK
EPAD = EPT * NT
BPB = 8
NBLK = NCH // BPB
RPT = NPAD // NT
RCH = 128
NRC = RPT // RCH


def _body(e_hbm, src_hbm, dst_hbm, w_hbm, out_hbm,
          xa, xb, src_b, dst_b, w_b, gbuf, bounce):
  c = lax.axis_index("c")
  s = lax.axis_index("s")
  base = s * RPT

  for i in range(NRC):
    r0 = base + i * RCH
    pltpu.sync_copy(e_hbm.at[c, pl.ds(r0, RCH)], bounce)
    pltpu.sync_copy(bounce, xa.at[pl.ds(r0, RCH)])
  plsc.subcore_barrier()

  for it in range(K):
    in_buf, out_buf = (xa, xb) if it % 2 == 0 else (xb, xa)

    for i in range(NRC):
      r0 = base + i * RCH
      pltpu.sync_copy(e_hbm.at[c, pl.ds(r0, RCH)], bounce)

      def _alpha_row(r):
        for q in range(4):
          bounce[r, pl.ds(q * 16, 16)] = bounce[r, pl.ds(q * 16, 16)] * ALPHA
      lax.fori_loop(0, RCH, lambda r, _: (_alpha_row(r), None)[1], None)
      pltpu.sync_copy(bounce, out_buf.at[pl.ds(r0, RCH)])
    plsc.subcore_barrier()

    def _block(b):
      pltpu.sync_copy(src_hbm.at[s, pl.ds(b * BPB, BPB)], src_b)
      pltpu.sync_copy(dst_hbm.at[s, pl.ds(b * BPB, BPB)], dst_b)
      pltpu.sync_copy(w_hbm.at[s, pl.ds(b * BPB, BPB)], w_b)

      def _scale_w(jj):
        for q in range(8):
          w_b[jj, pl.ds(q * 16, 16)] = w_b[jj, pl.ds(q * 16, 16)] * (1.0 - ALPHA)
      lax.fori_loop(0, BPB, lambda jj, _: (_scale_w(jj), None)[1], None)

      def _chunk(jj):
        pltpu.sync_copy(in_buf.at[src_b.at[jj]], gbuf)

        def _group(g):
          w16 = w_b[jj, pl.ds(g * 16, 16)]
          for el in range(16):
            e = g * 16 + el
            w = w16[el]
            for q in range(4):
              gbuf[e, pl.ds(q * 16, 16)] = gbuf[e, pl.ds(q * 16, 16)] * w
        lax.fori_loop(0, CHUNK // 16, lambda g, _: (_group(g), None)[1], None)

        pltpu.sync_copy(gbuf, out_buf.at[dst_b.at[jj]], add=True)
      lax.fori_loop(0, BPB, lambda jj, _: (_chunk(jj), None)[1], None)
    lax.fori_loop(0, NBLK, lambda b, _: (_block(b), None)[1], None)
    plsc.subcore_barrier()

  for i in range(NRC):
    r0 = base + i * RCH
    pltpu.sync_copy(xa.at[pl.ds(r0, RCH)], bounce)
    pltpu.sync_copy(bounce, out_hbm.at[c, pl.ds(r0, RCH)])


_sc_call = pl.kernel(
    _body,
    out_type=jax.ShapeDtypeStruct((2, NPAD, DH), jnp.float32),
    mesh=plsc.VectorSubcoreMesh(core_axis_name="c", subcore_axis_name="s"),
    compiler_params=pltpu.CompilerParams(use_tc_tiling_on_sc=False),
    scratch_types=[
        pltpu.VMEM_SHARED((NPAD, DH), jnp.float32),
        pltpu.VMEM_SHARED((NPAD, DH), jnp.float32),
        pltpu.VMEM((BPB, CHUNK), jnp.int32),
        pltpu.VMEM((BPB, CHUNK), jnp.int32),
        pltpu.VMEM((BPB, CHUNK), jnp.float32),
        pltpu.VMEM((CHUNK, DH), jnp.float32),
        pltpu.VMEM((RCH, DH), jnp.float32),
    ],
)


@jax.jit
def kernel(ent_embed, edge_index, edge_weight):
  dst = edge_index[0].astype(jnp.int32)
  src = edge_index[1].astype(jnp.int32)
  w = edge_weight.astype(jnp.float32)

  pad = EPAD - E
  src_p = jnp.pad(src, (0, pad)).reshape(NT, NCH, CHUNK)
  dst_p = jnp.pad(dst, (0, pad)).reshape(NT, NCH, CHUNK)
  w_p = jnp.pad(w, (0, pad)).reshape(NT, NCH, CHUNK)

  ep = jnp.pad(ent_embed, ((0, NPAD - N), (0, 0)))
  e_halves = ep.reshape(NPAD, 2, DH).transpose(1, 0, 2)

  out = _sc_call(e_halves, src_p, dst_p, w_p)
  return out.transpose(1, 0, 2).reshape(NPAD, D)[:N]
